# Initial kernel scaffold; baseline (speedup 1.0000x reference)
import jax
import jax.numpy as jnp
from jax.experimental import pallas as pl


def kernel(x_querry, g_p_0, g_p_1, e_p_2, e_k_2, e_p_3, e_k_3, e_p_4, e_k_4):
    raise NotImplementedError("write your pallas kernel here")


# speedup vs baseline: 1.8158x; 1.8158x over previous
import functools

import jax
import jax.numpy as jnp
from jax import lax
from jax.experimental import pallas as pl
from jax.experimental.pallas import tpu as pltpu
from jax.experimental.pallas import tpu_sc as plsc

_B = 128
_D = 768
_POOL = 36
_TOPK = 5
_EPL = 20
_HALF = _EPL // 2
_NSEL = _TOPK * _HALF
_ROWS = _B * _NSEL


def _tc_body(x_ref, k2_ref, k3_ref, rk2_ref, rv2_ref, rk3_ref, rv3_ref,
             loss_ref):
    x = x_ref[...]
    q = x / jnp.maximum(jnp.sqrt(jnp.sum(x * x, axis=1, keepdims=True)),
                        1e-12)
    c10 = lax.broadcasted_iota(jnp.int32, (_B, _NSEL), 1) // _HALF
    mod = lax.broadcasted_iota(jnp.int32, (_B, _NSEL), 1) % _HALF
    iota_p = lax.broadcasted_iota(jnp.int32, (_B, _POOL), 1).astype(
        jnp.float32)
    loss_sum = jnp.float32(0.0)
    for k_ref, rk_ref, rv_ref in ((k2_ref, rk2_ref, rv2_ref),
                                  (k3_ref, rk3_ref, rv3_ref)):
        kk = k_ref[...]
        nk = kk / jnp.maximum(
            jnp.sqrt(jnp.sum(kk * kk, axis=1, keepdims=True)), 1e-12)
        cos = lax.dot_general(q, nk, (((1,), (1,)), ((), ())),
                              preferred_element_type=jnp.float32)
        work = cos
        idxs = []
        for _ in range(_TOPK):
            m = jnp.max(work, axis=1, keepdims=True)
            idx = jnp.min(
                jnp.where(work == m, iota_p, jnp.float32(_POOL)),
                axis=1, keepdims=True)
            idxs.append(idx)
            work = jnp.where(iota_p == idx, -jnp.inf, work)
        colsum = jnp.sum(cos, axis=0, keepdims=True)
        sel = jnp.float32(0.0)
        rows_f = jnp.zeros((_B, _NSEL), jnp.float32)
        for t in range(_TOPK):
            sel = sel + jnp.sum(jnp.where(iota_p == idxs[t], colsum, 0.0))
            rows_f = rows_f + jnp.where(
                c10 == t, jnp.broadcast_to(idxs[t], (_B, _NSEL)), 0.0)
        rows_k = rows_f.astype(jnp.int32) * _EPL + mod
        rk_ref[...] = rows_k
        rv_ref[...] = rows_k + _HALF
        loss_sum = loss_sum + (1.0 - sel / jnp.float32(_B * _B * _TOPK))
    loss_ref[...] = jnp.full((8, 128), loss_sum / jnp.float32(3.0),
                             jnp.float32)


def _tc_select(x, k2, k3):
    return pl.pallas_call(
        _tc_body,
        out_shape=(
            jax.ShapeDtypeStruct((_B, _NSEL), jnp.int32),
            jax.ShapeDtypeStruct((_B, _NSEL), jnp.int32),
            jax.ShapeDtypeStruct((_B, _NSEL), jnp.int32),
            jax.ShapeDtypeStruct((_B, _NSEL), jnp.int32),
            jax.ShapeDtypeStruct((8, 128), jnp.float32),
        ),
    )(x, k2, k3)


def _sc_gather(p2, p3, rk2, rv2, rk3, rv3):
    info = plsc.get_sparse_core_info()
    nw = info.num_cores * info.num_subcores
    rpw = _ROWS // nw
    ch = 40
    nch = rpw // ch
    out_t = jax.ShapeDtypeStruct((_ROWS, _D), jnp.float32)
    mesh = plsc.VectorSubcoreMesh(core_axis_name="c", subcore_axis_name="s")

    @functools.partial(
        pl.kernel,
        mesh=mesh,
        out_type=[out_t, out_t, out_t, out_t],
        scratch_types=[
            pltpu.VMEM((ch,), jnp.int32),
            pltpu.VMEM((ch,), jnp.int32),
            pltpu.VMEM((2, ch, _D), jnp.float32),
            pltpu.SemaphoreType.DMA,
            pltpu.SemaphoreType.DMA,
            pltpu.SemaphoreType.DMA,
            pltpu.SemaphoreType.DMA,
        ],
    )
    def k(p2_h, p3_h, rk2_h, rv2_h, rk3_h, rv3_h,
          ok2, ov2, ok3, ov3, idx_a, idx_b, bufs, g0, g1, w0, w1):
        idxs = (idx_a, idx_b)
        gsems = (g0, g1)
        wsems = (w0, w1)
        wid = lax.axis_index("s") * info.num_cores + lax.axis_index("c")
        base = wid * rpw
        steps = []
        for pool, rows, out in ((p2_h, rk2_h, ok2), (p2_h, rv2_h, ov2),
                                (p3_h, rk3_h, ok3), (p3_h, rv3_h, ov3)):
            for c in range(nch):
                steps.append((pool, rows, out, c))
        n = len(steps)

        def start_gather(s, b):
            pool, rows, _, c = steps[s]
            pltpu.sync_copy(rows.at[pl.ds(base + c * ch, ch)], idxs[b])
            return pltpu.async_copy(pool.at[idxs[b]], bufs.at[b], gsems[b])

        def start_write(s, b):
            _, _, out, c = steps[s]
            return pltpu.async_copy(
                bufs.at[b], out.at[pl.ds(base + c * ch, ch)], wsems[b])

        g = [None, None]
        w = [None, None]
        g[0] = start_gather(0, 0)
        for s in range(n):
            b = s % 2
            nb = 1 - b
            if s + 1 < n:
                if w[nb] is not None:
                    w[nb].wait()
                g[nb] = start_gather(s + 1, nb)
            g[b].wait()
            w[b] = start_write(s, b)
        w[(n - 1) % 2].wait()

    return k(p2, p3, rk2, rv2, rk3, rv3)


def kernel(x_querry, g_p_0, g_p_1, e_p_2, e_k_2, e_p_3, e_k_3, e_p_4,
           e_k_4):
    del e_p_4, e_k_4
    rk2, rv2, rk3, rv3, loss2d = _tc_select(x_querry, e_k_2, e_k_3)
    ok2, ov2, ok3, ov3 = _sc_gather(
        e_p_2.reshape(_POOL * _EPL, _D), e_p_3.reshape(_POOL * _EPL, _D),
        rk2.reshape(-1), rv2.reshape(-1), rk3.reshape(-1), rv3.reshape(-1))
    half_g = 3
    pk0 = jnp.broadcast_to(g_p_0[None, :half_g, :], (_B, half_g, _D))
    pv0 = jnp.broadcast_to(g_p_0[None, half_g:, :], (_B, half_g, _D))
    pk1 = jnp.broadcast_to(g_p_1[None, :half_g, :], (_B, half_g, _D))
    pv1 = jnp.broadcast_to(g_p_1[None, half_g:, :], (_B, half_g, _D))
    return (pk0, pv0, pk1, pv1,
            ok2.reshape(_B, _NSEL, _D), ov2.reshape(_B, _NSEL, _D),
            ok3.reshape(_B, _NSEL, _D), ov3.reshape(_B, _NSEL, _D),
            loss2d[0, 0])


# speedup vs baseline: 1.8601x; 1.0244x over previous
import functools

import jax
import jax.numpy as jnp
from jax import lax
from jax.experimental import pallas as pl
from jax.experimental.pallas import tpu as pltpu
from jax.experimental.pallas import tpu_sc as plsc

_B = 128
_D = 768
_LANES = 128
_SUB = _D // _LANES
_POOL = 36
_TOPK = 5
_EPL = 20
_HALF = _EPL // 2
_NSEL = _TOPK * _HALF
_NSUB = _NSEL * _SUB
_ROWS = _B * _NSEL
_SROWS = _ROWS * _SUB
_PSUB = _POOL * _EPL * _SUB


def _tc_body(x_ref, k2_ref, k3_ref, rk2_ref, rv2_ref, rk3_ref, rv3_ref,
             loss_ref):
    x = x_ref[...]
    q = x / jnp.maximum(jnp.sqrt(jnp.sum(x * x, axis=1, keepdims=True)),
                        1e-12)
    seg = lax.broadcasted_iota(jnp.int32, (_B, _NSUB), 1) // (_HALF * _SUB)
    mod = lax.broadcasted_iota(jnp.int32, (_B, _NSUB), 1) % (_HALF * _SUB)
    iota_p = lax.broadcasted_iota(jnp.int32, (_B, _POOL), 1).astype(
        jnp.float32)
    loss_sum = jnp.float32(0.0)
    for k_ref, rk_ref, rv_ref in ((k2_ref, rk2_ref, rv2_ref),
                                  (k3_ref, rk3_ref, rv3_ref)):
        kk = k_ref[...]
        nk = kk / jnp.maximum(
            jnp.sqrt(jnp.sum(kk * kk, axis=1, keepdims=True)), 1e-12)
        cos = lax.dot_general(q, nk, (((1,), (1,)), ((), ())),
                              preferred_element_type=jnp.float32)
        work = cos
        idxs = []
        for _ in range(_TOPK):
            m = jnp.max(work, axis=1, keepdims=True)
            idx = jnp.min(
                jnp.where(work == m, iota_p, jnp.float32(_POOL)),
                axis=1, keepdims=True)
            idxs.append(idx)
            work = jnp.where(iota_p == idx, -jnp.inf, work)
        colsum = jnp.sum(cos, axis=0, keepdims=True)
        sel = jnp.float32(0.0)
        rows_f = jnp.zeros((_B, _NSUB), jnp.float32)
        for t in range(_TOPK):
            sel = sel + jnp.sum(jnp.where(iota_p == idxs[t], colsum, 0.0))
            rows_f = rows_f + jnp.where(
                seg == t, jnp.broadcast_to(idxs[t], (_B, _NSUB)), 0.0)
        rows_k = rows_f.astype(jnp.int32) * (_EPL * _SUB) + mod
        rk_ref[...] = rows_k
        rv_ref[...] = rows_k + _HALF * _SUB
        loss_sum = loss_sum + (1.0 - sel / jnp.float32(_B * _B * _TOPK))
    loss_ref[...] = jnp.full((8, 128), loss_sum / jnp.float32(3.0),
                             jnp.float32)


def _tc_select(x, k2, k3):
    return pl.pallas_call(
        _tc_body,
        out_shape=(
            jax.ShapeDtypeStruct((_B, _NSUB), jnp.int32),
            jax.ShapeDtypeStruct((_B, _NSUB), jnp.int32),
            jax.ShapeDtypeStruct((_B, _NSUB), jnp.int32),
            jax.ShapeDtypeStruct((_B, _NSUB), jnp.int32),
            jax.ShapeDtypeStruct((8, 128), jnp.float32),
        ),
    )(x, k2, k3)


def _sc_gather(pool, rk, rv):
    info = plsc.get_sparse_core_info()
    nw = info.num_cores * info.num_subcores
    rpw = _ROWS // nw
    ch = 40
    nch = rpw // ch
    chs = ch * _SUB
    out_t = jax.ShapeDtypeStruct((_SROWS, _LANES), jnp.float32)
    mesh = plsc.VectorSubcoreMesh(core_axis_name="c", subcore_axis_name="s")

    @functools.partial(
        pl.kernel,
        mesh=mesh,
        out_type=[out_t, out_t],
        scratch_types=[
            pltpu.VMEM((chs,), jnp.int32),
            pltpu.VMEM((chs,), jnp.int32),
            pltpu.VMEM((2, chs, _LANES), jnp.float32),
            pltpu.SemaphoreType.DMA,
            pltpu.SemaphoreType.DMA,
            pltpu.SemaphoreType.DMA,
            pltpu.SemaphoreType.DMA,
        ],
    )
    def k(pool_h, rk_h, rv_h, ok, ov, idx_a, idx_b, bufs, g0, g1, w0, w1):
        idxs = (idx_a, idx_b)
        gsems = (g0, g1)
        wsems = (w0, w1)
        wid = lax.axis_index("s") * info.num_cores + lax.axis_index("c")
        base = wid * rpw * _SUB
        steps = []
        for rows, out in ((rk_h, ok), (rv_h, ov)):
            for c in range(nch):
                steps.append((rows, out, c))
        n = len(steps)

        def start_gather(s, b):
            rows, _, c = steps[s]
            pltpu.sync_copy(rows.at[pl.ds(base + c * chs, chs)], idxs[b])
            return pltpu.async_copy(pool_h.at[idxs[b]], bufs.at[b],
                                    gsems[b])

        def start_write(s, b):
            _, out, c = steps[s]
            return pltpu.async_copy(
                bufs.at[b], out.at[pl.ds(base + c * chs, chs)], wsems[b])

        g = [None, None]
        w = [None, None]
        g[0] = start_gather(0, 0)
        for s in range(n):
            b = s % 2
            nb = 1 - b
            if s + 1 < n:
                if w[nb] is not None:
                    w[nb].wait()
                g[nb] = start_gather(s + 1, nb)
            g[b].wait()
            w[b] = start_write(s, b)
        w[(n - 1) % 2].wait()

    return k(pool, rk, rv)


def kernel(x_querry, g_p_0, g_p_1, e_p_2, e_k_2, e_p_3, e_k_3, e_p_4,
           e_k_4):
    del e_p_4, e_k_4
    rk2, rv2, rk3, rv3, loss2d = _tc_select(x_querry, e_k_2, e_k_3)
    ok2, ov2 = _sc_gather(e_p_2.reshape(_PSUB, _LANES),
                          rk2.reshape(-1), rv2.reshape(-1))
    ok3, ov3 = _sc_gather(e_p_3.reshape(_PSUB, _LANES),
                          rk3.reshape(-1), rv3.reshape(-1))
    half_g = 3
    pk0 = jnp.broadcast_to(g_p_0[None, :half_g, :], (_B, half_g, _D))
    pv0 = jnp.broadcast_to(g_p_0[None, half_g:, :], (_B, half_g, _D))
    pk1 = jnp.broadcast_to(g_p_1[None, :half_g, :], (_B, half_g, _D))
    pv1 = jnp.broadcast_to(g_p_1[None, half_g:, :], (_B, half_g, _D))
    return (pk0, pv0, pk1, pv1,
            ok2.reshape(_B, _NSEL, _D), ov2.reshape(_B, _NSEL, _D),
            ok3.reshape(_B, _NSEL, _D), ov3.reshape(_B, _NSEL, _D),
            loss2d[0, 0])
